# Initial kernel scaffold; baseline (speedup 1.0000x reference)
import jax
import jax.numpy as jnp
from jax.experimental import pallas as pl


def kernel(z_number, nbrs, r_ij, embed_table, W_dist, b_dist, W_comb, b_comb):
    raise NotImplementedError("write your pallas kernel here")


# speedup vs baseline: 1.6157x; 1.6157x over previous
import functools

import jax
import jax.numpy as jnp
from jax import lax
from jax.experimental import pallas as pl
from jax.experimental.pallas import tpu as pltpu
from jax.experimental.pallas import tpu_sc as plsc

_EPS = 1e-15
_CUTOFF = 5.0


def _w_filter(r_ij, W_dist, b_dist2, *, n_rbf, feat, be):
    n_edges = r_ij.shape[0]
    delta = _CUTOFF / (n_rbf - 1)
    coeff = -0.5 / delta**2

    def body(r_ref, wd_ref, bd_ref, out_ref):
        rr = r_ref[...]
        d2 = jnp.sum(rr * rr, axis=1, keepdims=True) + 3.0 * _EPS
        d = jnp.sqrt(d2)
        offs = lax.broadcasted_iota(jnp.int32, (be, n_rbf), 1).astype(
            jnp.float32) * delta
        e = jnp.exp(coeff * (d - offs) ** 2)
        w = jnp.dot(e, wd_ref[...], preferred_element_type=jnp.float32)
        w = w + bd_ref[...]
        c = 0.5 * (jnp.cos(jnp.pi / _CUTOFF * d) + 1.0)
        c = jnp.where(d < _CUTOFF, c, 0.0)
        out_ref[...] = w * c

    return pl.pallas_call(
        body,
        grid=(n_edges // be,),
        in_specs=[
            pl.BlockSpec((be, 3), lambda i: (i, 0)),
            pl.BlockSpec((n_rbf, feat), lambda i: (0, 0)),
            pl.BlockSpec((1, feat), lambda i: (0, 0)),
        ],
        out_specs=pl.BlockSpec((be, feat), lambda i: (i, 0)),
        out_shape=jax.ShapeDtypeStruct((n_edges, feat), jnp.float32),
    )(r_ij, W_dist, b_dist2)


def _x_lookup(z2, et_pad, *, n_atoms, feat):
    def body(z_ref, et_ref, out_ref):
        z = z_ref[...]
        ids = lax.broadcasted_iota(jnp.int32, (n_atoms, feat), 1)
        onehot = (z == ids).astype(jnp.float32)
        out_ref[...] = jnp.dot(onehot, et_ref[...],
                               preferred_element_type=jnp.float32)

    return pl.pallas_call(
        body,
        grid=(1,),
        in_specs=[
            pl.BlockSpec((n_atoms, 1), lambda i: (0, 0)),
            pl.BlockSpec((feat, feat), lambda i: (0, 0)),
        ],
        out_specs=pl.BlockSpec((n_atoms, feat), lambda i: (0, 0)),
        out_shape=jax.ShapeDtypeStruct((n_atoms, feat), jnp.float32),
    )(z2, et_pad)


def _sc_aggregate(dst, src, x, w, *, n_atoms, feat):
    n_edges = dst.shape[0]
    CH = 128
    n_chunks = n_edges // CH
    NW = 32
    base_chunks = n_chunks // NW
    extra = n_chunks - base_chunks * NW
    rpt = (n_atoms // 16) // 8 * 8
    rpt_last = n_atoms - 15 * rpt
    mesh = plsc.VectorSubcoreMesh(core_axis_name="c", subcore_axis_name="s")

    @functools.partial(
        pl.kernel,
        out_type=jax.ShapeDtypeStruct((2, n_atoms, feat), jnp.float32),
        mesh=mesh,
        scratch_types=[
            pltpu.VMEM((1, CH), jnp.int32),
            pltpu.VMEM((1, CH), jnp.int32),
            pltpu.VMEM((CH, feat), jnp.float32),
            pltpu.VMEM((CH, feat), jnp.float32),
            pltpu.VMEM_SHARED((n_atoms, feat), jnp.float32),
            pltpu.SemaphoreType.DMA,
        ],
    )
    def sc_kernel(dst_hbm, src_hbm, x_hbm, w_hbm, out_hbm,
                  dsti, srci, rows, wbuf, agg_sh, sem):
        cid = lax.axis_index("c")
        sid = lax.axis_index("s")
        wid = cid * 16 + sid

        @pl.loop(0, CH)
        def _(r):
            for cb in range(feat // 16):
                rows[r, pl.ds(cb * 16, 16)] = jnp.zeros((16,), jnp.float32)

        nz16 = jnp.where(sid == 15, rpt_last // 16, rpt // 16)

        @pl.loop(0, nz16)
        def _(p):
            pltpu.sync_copy(rows.at[pl.ds(0, 16)],
                            agg_sh.at[pl.ds(sid * rpt + p * 16, 16)])
        plsc.subcore_barrier()

        nch = base_chunks + jnp.where(wid < extra, 1, 0)
        start = wid * base_chunks + jnp.minimum(wid, extra)

        @pl.loop(0, nch)
        def _(j):
            base = (start + j) * CH
            pltpu.sync_copy(dst_hbm.at[pl.ds(base, CH)], dsti.at[0])
            pltpu.sync_copy(src_hbm.at[pl.ds(base, CH)], srci.at[0])
            pltpu.async_copy(x_hbm.at[dsti.at[0]], rows, sem).wait()
            pltpu.sync_copy(w_hbm.at[pl.ds(base, CH)], wbuf)

            @pl.loop(0, CH)
            def _(r):
                for cb in range(feat // 16):
                    sl = pl.ds(cb * 16, 16)
                    wbuf[r, sl] = wbuf[r, sl] * rows[r, sl]

            pltpu.sync_copy(wbuf, agg_sh.at[srci.at[0]], add=True)

        plsc.subcore_barrier()

        @pl.when(sid == 15)
        def _():
            pltpu.sync_copy(
                agg_sh.at[pl.ds(15 * rpt, rpt_last)],
                out_hbm.at[cid, pl.ds(15 * rpt, rpt_last)])

        @pl.when(sid != 15)
        def _():
            pltpu.sync_copy(
                agg_sh.at[pl.ds(sid * rpt, rpt)],
                out_hbm.at[cid, pl.ds(sid * rpt, rpt)])

    return sc_kernel(dst, src, x, w)


def _combine(x, aggs, W_comb, b_comb2, *, n_atoms, feat, bn):
    def body(x_ref, a_ref, wc_ref, bc_ref, out_ref):
        agg = a_ref[0] + a_ref[1]
        s = jnp.dot(x_ref[...], wc_ref[0:feat, :],
                    preferred_element_type=jnp.float32)
        s = s + jnp.dot(agg, wc_ref[feat:2 * feat, :],
                        preferred_element_type=jnp.float32)
        out_ref[...] = s + bc_ref[...]

    return pl.pallas_call(
        body,
        grid=(n_atoms // bn,),
        in_specs=[
            pl.BlockSpec((bn, feat), lambda i: (i, 0)),
            pl.BlockSpec((2, bn, feat), lambda i: (0, i, 0)),
            pl.BlockSpec((2 * feat, feat), lambda i: (0, 0)),
            pl.BlockSpec((1, feat), lambda i: (0, 0)),
        ],
        out_specs=pl.BlockSpec((bn, feat), lambda i: (i, 0)),
        out_shape=jax.ShapeDtypeStruct((n_atoms, feat), jnp.float32),
    )(x, aggs, W_comb, b_comb2)


def kernel(z_number, nbrs, r_ij, embed_table, W_dist, b_dist, W_comb, b_comb):
    n_atoms = z_number.shape[0]
    feat = embed_table.shape[1]
    n_rbf = W_dist.shape[0]

    src = nbrs[:, 0]
    dst = nbrs[:, 1]
    et_pad = jnp.pad(embed_table, ((0, feat - embed_table.shape[0]), (0, 0)))
    z2 = z_number.reshape(-1, 1).astype(jnp.int32)

    x = _x_lookup(z2, et_pad, n_atoms=n_atoms, feat=feat)
    w = _w_filter(r_ij, W_dist, b_dist.reshape(1, -1),
                  n_rbf=n_rbf, feat=feat, be=2000)
    aggs = _sc_aggregate(dst, src, x, w, n_atoms=n_atoms, feat=feat)
    s = _combine(x, aggs, W_comb, b_comb.reshape(1, -1),
                 n_atoms=n_atoms, feat=feat, bn=2000)
    v = jnp.zeros((n_atoms, feat, 3), jnp.float32)
    return (s, v)


# speedup vs baseline: 3.0362x; 1.8792x over previous
import functools

import jax
import jax.numpy as jnp
from jax import lax
from jax.experimental import pallas as pl
from jax.experimental.pallas import tpu as pltpu
from jax.experimental.pallas import tpu_sc as plsc

_EPS = 1e-15
_CUTOFF = 5.0


_COSPI_SQRT_COEFFS = None


def _cospi_sqrt_coeffs(deg=12):
    global _COSPI_SQRT_COEFFS
    if _COSPI_SQRT_COEFFS is None:
        import math
        c = []
        for k in range(deg + 1):
            c.append((-1.0) ** k * math.pi ** (2 * k) / math.factorial(2 * k))
        _COSPI_SQRT_COEFFS = c
    return _COSPI_SQRT_COEFFS


def _w_filter(r_ijT3, wd_aug, *, n_rbf, feat, be):
    n_edges = r_ijT3.shape[0] * be
    delta = _CUTOFF / (n_rbf - 1)
    coeff = -0.5 / delta**2

    def body(r_ref, wd_ref, out_ref):
        rr = r_ref[0]
        d2 = jnp.sum(rr * rr, axis=0, keepdims=True) + 3.0 * _EPS
        d = jnp.sqrt(d2)
        t = jnp.minimum(d2 * (1.0 / _CUTOFF**2), 1.0)
        cf = _cospi_sqrt_coeffs()
        cp = jnp.full_like(t, cf[-1])
        for a in reversed(cf[:-1]):
            cp = cp * t + a
        c = 0.5 * (cp + 1.0)
        c = jnp.where(d < _CUTOFF, c, 0.0)
        offs = lax.broadcasted_iota(jnp.int32, (n_rbf + 1, be), 0).astype(
            jnp.float32) * delta
        e = jnp.exp(coeff * (d - offs) ** 2)
        ones = jnp.ones((1, be), jnp.float32)
        e = jnp.concatenate([e[:n_rbf], ones], axis=0)
        ec = e * c
        out_ref[...] = lax.dot_general(
            ec, wd_ref[...], (((0,), (0,)), ((), ())),
            preferred_element_type=jnp.float32)

    return pl.pallas_call(
        body,
        grid=(n_edges // be,),
        in_specs=[
            pl.BlockSpec((1, 3, be), lambda i: (i, 0, 0)),
            pl.BlockSpec((n_rbf + 1, feat), lambda i: (0, 0)),
        ],
        out_specs=pl.BlockSpec((be, feat), lambda i: (i, 0)),
        out_shape=jax.ShapeDtypeStruct((n_edges, feat), jnp.float32),
    )(r_ijT3, wd_aug)


def _x_lookup(z2, et_pad, *, n_atoms, feat):
    def body(z_ref, et_ref, out_ref):
        z = z_ref[...]
        ids = lax.broadcasted_iota(jnp.int32, (n_atoms, feat), 1)
        onehot = (z == ids).astype(jnp.float32)
        out_ref[...] = jnp.dot(onehot, et_ref[...],
                               preferred_element_type=jnp.float32)

    return pl.pallas_call(
        body,
        grid=(1,),
        in_specs=[
            pl.BlockSpec((n_atoms, 1), lambda i: (0, 0)),
            pl.BlockSpec((feat, feat), lambda i: (0, 0)),
        ],
        out_specs=pl.BlockSpec((n_atoms, feat), lambda i: (0, 0)),
        out_shape=jax.ShapeDtypeStruct((n_atoms, feat), jnp.float32),
    )(z2, et_pad)


def _sc_aggregate(dst, src, x, w, *, n_atoms, feat):
    n_edges = dst.shape[0]
    CH = 128
    n_chunks = n_edges // CH
    NW = 32
    base_chunks = n_chunks // NW
    extra = n_chunks - base_chunks * NW
    rpt = (n_atoms // 16) // 8 * 8
    rpt_last = n_atoms - 15 * rpt
    mesh = plsc.VectorSubcoreMesh(core_axis_name="c", subcore_axis_name="s")

    @functools.partial(
        pl.kernel,
        out_type=jax.ShapeDtypeStruct((2, n_atoms, feat), jnp.float32),
        mesh=mesh,
        scratch_types=[
            pltpu.VMEM((1, CH), jnp.int32),
            pltpu.VMEM((1, CH), jnp.int32),
            pltpu.VMEM((CH, feat), jnp.float32),
            pltpu.VMEM((CH, feat), jnp.float32),
            pltpu.VMEM_SHARED((n_atoms, feat), jnp.float32),
            pltpu.SemaphoreType.DMA,
        ],
    )
    def sc_kernel(dst_hbm, src_hbm, x_hbm, w_hbm, out_hbm,
                  dsti, srci, rows, wbuf, agg_sh, sem):
        cid = lax.axis_index("c")
        sid = lax.axis_index("s")
        wid = cid * 16 + sid

        @pl.loop(0, CH)
        def _(r):
            for cb in range(feat // 16):
                rows[r, pl.ds(cb * 16, 16)] = jnp.zeros((16,), jnp.float32)

        nz16 = jnp.where(sid == 15, rpt_last // 16, rpt // 16)

        @pl.loop(0, nz16)
        def _(p):
            pltpu.sync_copy(rows.at[pl.ds(0, 16)],
                            agg_sh.at[pl.ds(sid * rpt + p * 16, 16)])
        plsc.subcore_barrier()

        nch = base_chunks + jnp.where(wid < extra, 1, 0)
        start = wid * base_chunks + jnp.minimum(wid, extra)

        @pl.loop(0, nch)
        def _(j):
            base = (start + j) * CH
            pltpu.sync_copy(dst_hbm.at[pl.ds(base, CH)], dsti.at[0])
            pltpu.sync_copy(src_hbm.at[pl.ds(base, CH)], srci.at[0])
            pltpu.async_copy(x_hbm.at[dsti.at[0]], rows, sem).wait()
            pltpu.sync_copy(w_hbm.at[pl.ds(base, CH)], wbuf)

            @pl.loop(0, CH)
            def _(r):
                for cb in range(feat // 16):
                    sl = pl.ds(cb * 16, 16)
                    wbuf[r, sl] = wbuf[r, sl] * rows[r, sl]

            pltpu.sync_copy(wbuf, agg_sh.at[srci.at[0]], add=True)

        plsc.subcore_barrier()

        @pl.when(sid == 15)
        def _():
            pltpu.sync_copy(
                agg_sh.at[pl.ds(15 * rpt, rpt_last)],
                out_hbm.at[cid, pl.ds(15 * rpt, rpt_last)])

        @pl.when(sid != 15)
        def _():
            pltpu.sync_copy(
                agg_sh.at[pl.ds(sid * rpt, rpt)],
                out_hbm.at[cid, pl.ds(sid * rpt, rpt)])

    return sc_kernel(dst, src, x, w)


def _combine(x, aggs, W_comb, b_comb2, *, n_atoms, feat, bn):
    def body(x_ref, a_ref, wc_ref, bc_ref, out_ref):
        agg = a_ref[0] + a_ref[1]
        s = jnp.dot(x_ref[...], wc_ref[0:feat, :],
                    preferred_element_type=jnp.float32)
        s = s + jnp.dot(agg, wc_ref[feat:2 * feat, :],
                        preferred_element_type=jnp.float32)
        out_ref[...] = s + bc_ref[...]

    return pl.pallas_call(
        body,
        grid=(n_atoms // bn,),
        in_specs=[
            pl.BlockSpec((bn, feat), lambda i: (i, 0)),
            pl.BlockSpec((2, bn, feat), lambda i: (0, i, 0)),
            pl.BlockSpec((2 * feat, feat), lambda i: (0, 0)),
            pl.BlockSpec((1, feat), lambda i: (0, 0)),
        ],
        out_specs=pl.BlockSpec((bn, feat), lambda i: (i, 0)),
        out_shape=jax.ShapeDtypeStruct((n_atoms, feat), jnp.float32),
    )(x, aggs, W_comb, b_comb2)


def kernel(z_number, nbrs, r_ij, embed_table, W_dist, b_dist, W_comb, b_comb):
    n_atoms = z_number.shape[0]
    feat = embed_table.shape[1]
    n_rbf = W_dist.shape[0]

    src = nbrs[:, 0]
    dst = nbrs[:, 1]
    et_pad = jnp.pad(embed_table, ((0, feat - embed_table.shape[0]), (0, 0)))
    z2 = z_number.reshape(-1, 1).astype(jnp.int32)

    x = _x_lookup(z2, et_pad, n_atoms=n_atoms, feat=feat)
    wd_aug = jnp.concatenate([W_dist, b_dist.reshape(1, -1)], axis=0)
    be = 2000
    r_ijT3 = jnp.transpose(r_ij.T.reshape(3, -1, be), (1, 0, 2))
    w = _w_filter(r_ijT3, wd_aug, n_rbf=n_rbf, feat=feat, be=be)
    aggs = _sc_aggregate(dst, src, x, w, n_atoms=n_atoms, feat=feat)
    s = _combine(x, aggs, W_comb, b_comb.reshape(1, -1),
                 n_atoms=n_atoms, feat=feat, bn=2000)
    v = jnp.zeros((n_atoms, feat, 3), jnp.float32)
    return (s, v)
